# Initial kernel scaffold; baseline (speedup 1.0000x reference)
import jax
import jax.numpy as jnp
from jax.experimental import pallas as pl


def kernel(gender, age, occupation, gender_table, age_table, occ_table, W1, b1, W2, b2):
    raise NotImplementedError("write your pallas kernel here")


# speedup vs baseline: 4.2958x; 4.2958x over previous
import functools

import jax
import jax.numpy as jnp
from jax import lax
from jax.experimental import pallas as pl
from jax.experimental.pallas import tpu as pltpu
from jax.experimental.pallas import tpu_sc as plsc

_EMB = 32
_HID = 128
_B = 16384
_NG, _NA, _NO = 2, 7, 21
_NCOMBO = _NG * _NA * _NO
_NPAD = 304

_NC, _NS, _L = 2, 16, 16
_NW = _NC * _NS
_BPW = _B // _NW
_CHUNK = 128


def _table_body(gt_ref, at_ref, ot_ref, w1_ref, b1_ref, w2_ref, b2_ref, out_ref):
    r = lax.broadcasted_iota(jnp.int32, (_NPAD, 1), 0)
    g = r // (_NA * _NO)
    a = (r // _NO) % _NA
    o = r % _NO
    og = (g == lax.broadcasted_iota(jnp.int32, (_NPAD, _NG), 1)).astype(jnp.float32)
    oa = (a == lax.broadcasted_iota(jnp.int32, (_NPAD, _NA), 1)).astype(jnp.float32)
    oo = (o == lax.broadcasted_iota(jnp.int32, (_NPAD, _NO), 1)).astype(jnp.float32)
    xg = jnp.dot(og, gt_ref[...], preferred_element_type=jnp.float32)
    xa = jnp.dot(oa, at_ref[...], preferred_element_type=jnp.float32)
    xo = jnp.dot(oo, ot_ref[...], preferred_element_type=jnp.float32)
    x = jnp.concatenate([xg, xa, xo], axis=1)
    h = jnp.dot(x, w1_ref[...], preferred_element_type=jnp.float32) + b1_ref[...]
    h = jnp.maximum(h, 0.0)
    out = jnp.dot(h, w2_ref[...], preferred_element_type=jnp.float32) + b2_ref[...]
    norm = jnp.sqrt(jnp.sum(out * out, axis=1, keepdims=True))
    out_ref[...] = out / jnp.maximum(norm, 1e-12)


_combo_table = pl.pallas_call(
    _table_body,
    out_shape=jax.ShapeDtypeStruct((_NPAD, _EMB), jnp.float32),
)


def _sc_gather_body(g_hbm, a_hbm, o_hbm, tbl_hbm, out_hbm,
                    g_v, a_v, o_v, idx_v, rows_v, sem):
    wid = lax.axis_index("s") * _NC + lax.axis_index("c")
    base = wid * _BPW
    pltpu.sync_copy(g_hbm.at[pl.ds(base, _BPW)], g_v)
    pltpu.sync_copy(a_hbm.at[pl.ds(base, _BPW)], a_v)
    pltpu.sync_copy(o_hbm.at[pl.ds(base, _BPW)], o_v)

    def body(i, carry):
        s = pl.ds(i * _L, _L)
        idx_v[s] = g_v[s] * (_NA * _NO) + a_v[s] * _NO + o_v[s]
        return carry

    lax.fori_loop(0, _BPW // _L, body, 0)

    copies = [
        pltpu.async_copy(
            tbl_hbm.at[idx_v.at[pl.ds(c * _CHUNK, _CHUNK)]],
            rows_v.at[pl.ds(c * _CHUNK, _CHUNK)],
            sem,
        )
        for c in range(_BPW // _CHUNK)
    ]
    for cp in copies:
        cp.wait()
    pltpu.sync_copy(rows_v, out_hbm.at[pl.ds(base, _BPW)])


@functools.cache
def _make_sc_gather():
    mesh = plsc.VectorSubcoreMesh(core_axis_name="c", subcore_axis_name="s")
    return pl.kernel(
        _sc_gather_body,
        out_type=jax.ShapeDtypeStruct((_B, _EMB), jnp.float32),
        mesh=mesh,
        compiler_params=pltpu.CompilerParams(use_tc_tiling_on_sc=False),
        scratch_types=[
            pltpu.VMEM((_BPW,), jnp.int32),
            pltpu.VMEM((_BPW,), jnp.int32),
            pltpu.VMEM((_BPW,), jnp.int32),
            pltpu.VMEM((_BPW,), jnp.int32),
            pltpu.VMEM((_BPW, _EMB), jnp.float32),
            pltpu.SemaphoreType.DMA,
        ],
    )


def kernel(gender, age, occupation, gender_table, age_table, occ_table,
           W1, b1, W2, b2):
    tbl = _combo_table(
        gender_table, age_table, occ_table,
        W1, b1.reshape(1, _HID), W2, b2.reshape(1, _EMB),
    )
    return _make_sc_gather()(
        gender.astype(jnp.int32),
        age.astype(jnp.int32),
        occupation.astype(jnp.int32),
        tbl,
    )
